# baseline (speedup 1.0000x reference)
import functools

import jax
import jax.numpy as jnp
from jax import lax
from jax.experimental import pallas as pl
from jax.experimental.pallas import tpu as pltpu
from jax.experimental.pallas import tpu_sc as plsc

_VB = 4096


def _gather_fn(nc, ns, b_per_w, table_hbm, idx_hbm, out_hbm, idx_v, rows_v, sem):
    wid = lax.axis_index("s") * nc + lax.axis_index("c")
    base = wid * b_per_w
    pltpu.sync_copy(idx_hbm.at[pl.ds(base, b_per_w)], idx_v)
    pltpu.async_copy(table_hbm.at[idx_v], rows_v, sem).wait()
    pltpu.sync_copy(rows_v, out_hbm.at[pl.ds(base, b_per_w)])


def _sc_gather(table, idx_flat):
    info = plsc.get_sparse_core_info()
    nc, ns = info.num_cores, info.num_subcores
    nw = nc * ns
    b_total, d = idx_flat.shape[0], table.shape[1]
    b_per_w = b_total // nw
    mesh = plsc.VectorSubcoreMesh(core_axis_name="c", subcore_axis_name="s")
    kern = functools.partial(
        pl.kernel,
        mesh=mesh,
        out_type=jax.ShapeDtypeStruct((b_total, d), jnp.float32),
        scratch_types=[
            pltpu.VMEM((b_per_w,), jnp.int32),
            pltpu.VMEM((b_per_w, d), jnp.float32),
            pltpu.SemaphoreType.DMA,
        ],
        compiler_params=pltpu.CompilerParams(use_tc_tiling_on_sc=False),
    )(functools.partial(_gather_fn, nc, ns, b_per_w))
    return kern(table, idx_flat)


def _proj_body(h_ref, w_ref, b_ref, o_ref):
    o_ref[...] = (
        jnp.dot(h_ref[...], w_ref[...], preferred_element_type=jnp.float32)
        + b_ref[...]
    )


def _tc_project(hidden, proj_w, proj_b2d):
    r, h = hidden.shape
    v = proj_w.shape[1]
    grid = (pl.cdiv(v, _VB),)
    return pl.pallas_call(
        _proj_body,
        grid=grid,
        in_specs=[
            pl.BlockSpec((r, h), lambda j: (0, 0)),
            pl.BlockSpec((h, _VB), lambda j: (0, j)),
            pl.BlockSpec((1, _VB), lambda j: (0, j)),
        ],
        out_specs=pl.BlockSpec((r, _VB), lambda j: (0, j)),
        out_shape=jax.ShapeDtypeStruct((r, v), jnp.float32),
    )(hidden, proj_w, proj_b2d)


def kernel(input_ids, emb_table, proj_w, proj_b):
    b, q = input_ids.shape
    v = proj_w.shape[1]
    idx_flat = input_ids.reshape(b * q).astype(jnp.int32)
    hidden = _sc_gather(emb_table, idx_flat)
    logits = _tc_project(hidden, proj_w, proj_b.reshape(1, v))
    return logits.reshape(b, q, v)


# speedup vs baseline: 1.0057x; 1.0057x over previous
import functools

import jax
import jax.numpy as jnp
from jax import lax
from jax.experimental import pallas as pl
from jax.experimental.pallas import tpu as pltpu
from jax.experimental.pallas import tpu_sc as plsc

_VB = 4096
_NBUF = 4


def _gather_fn(nc, ns, b_per_w, table_hbm, idx_hbm, out_hbm, idx_v, rows_v, sem):
    wid = lax.axis_index("s") * nc + lax.axis_index("c")
    base = wid * b_per_w
    pltpu.sync_copy(idx_hbm.at[pl.ds(base, b_per_w)], idx_v)
    pltpu.async_copy(table_hbm.at[idx_v], rows_v, sem).wait()
    pltpu.sync_copy(rows_v, out_hbm.at[pl.ds(base, b_per_w)])


def _sc_gather(table, idx_flat):
    info = plsc.get_sparse_core_info()
    nc, ns = info.num_cores, info.num_subcores
    nw = nc * ns
    b_total, d = idx_flat.shape[0], table.shape[1]
    b_per_w = b_total // nw
    mesh = plsc.VectorSubcoreMesh(core_axis_name="c", subcore_axis_name="s")
    kern = functools.partial(
        pl.kernel,
        mesh=mesh,
        out_type=jax.ShapeDtypeStruct((b_total, d), jnp.float32),
        scratch_types=[
            pltpu.VMEM((b_per_w,), jnp.int32),
            pltpu.VMEM((b_per_w, d), jnp.float32),
            pltpu.SemaphoreType.DMA,
        ],
        compiler_params=pltpu.CompilerParams(use_tc_tiling_on_sc=False),
    )(functools.partial(_gather_fn, nc, ns, b_per_w))
    return kern(table, idx_flat)


def _main_body(ng, h_ref, w_ref, b_ref, out_hbm, acc, sems):
    g = pl.program_id(0)
    slot = lax.rem(g, _NBUF)

    @pl.when(g >= _NBUF)
    def _wait_prev():
        pltpu.make_async_copy(
            acc.at[slot],
            out_hbm.at[:, pl.ds((g - _NBUF) * _VB, _VB)],
            sems.at[slot],
        ).wait()

    acc[slot] = (
        jnp.dot(h_ref[...], w_ref[...], preferred_element_type=jnp.float32)
        + b_ref[...]
    )
    pltpu.make_async_copy(
        acc.at[slot], out_hbm.at[:, pl.ds(g * _VB, _VB)], sems.at[slot]
    ).start()

    @pl.when(g == ng - 1)
    def _drain():
        for k in range(_NBUF):
            gp = ng - _NBUF + k
            pltpu.make_async_copy(
                acc.at[gp % _NBUF],
                out_hbm.at[:, pl.ds(gp * _VB, _VB)],
                sems.at[gp % _NBUF],
            ).wait()


def _tail_body(h_ref, w_ref, b_ref, big_ref, o_ref):
    del big_ref
    o_ref[...] = (
        jnp.dot(h_ref[...], w_ref[...], preferred_element_type=jnp.float32)
        + b_ref[...]
    )


def _tc_project(hidden, proj_w, proj_b2d):
    r, h = hidden.shape
    v = proj_w.shape[1]
    ng_main = v // _VB
    main = pl.pallas_call(
        functools.partial(_main_body, ng_main),
        grid=(ng_main,),
        in_specs=[
            pl.BlockSpec((r, h), lambda g: (0, 0)),
            pl.BlockSpec((h, _VB), lambda g: (0, g)),
            pl.BlockSpec((1, _VB), lambda g: (0, g)),
        ],
        out_specs=pl.BlockSpec(memory_space=pl.ANY),
        out_shape=jax.ShapeDtypeStruct((r, v), jnp.float32),
        scratch_shapes=[
            pltpu.VMEM((_NBUF, r, _VB), jnp.float32),
            pltpu.SemaphoreType.DMA((_NBUF,)),
        ],
    )(hidden, proj_w, proj_b2d)
    return pl.pallas_call(
        _tail_body,
        grid=(1,),
        in_specs=[
            pl.BlockSpec((r, h), lambda g: (0, 0)),
            pl.BlockSpec((h, _VB), lambda g: (0, ng_main)),
            pl.BlockSpec((1, _VB), lambda g: (0, ng_main)),
            pl.BlockSpec(memory_space=pl.ANY),
        ],
        out_specs=pl.BlockSpec((r, _VB), lambda g: (0, ng_main)),
        out_shape=jax.ShapeDtypeStruct((r, v), jnp.float32),
        input_output_aliases={3: 0},
    )(hidden, proj_w, proj_b2d, main)


def kernel(input_ids, emb_table, proj_w, proj_b):
    b, q = input_ids.shape
    v = proj_w.shape[1]
    idx_flat = input_ids.reshape(b * q).astype(jnp.int32)
    hidden = _sc_gather(emb_table, idx_flat)
    logits = _tc_project(hidden, proj_w, proj_b.reshape(1, v))
    return logits.reshape(b, q, v)


# speedup vs baseline: 1.0191x; 1.0133x over previous
import functools

import jax
import jax.numpy as jnp
from jax import lax
from jax.experimental import pallas as pl
from jax.experimental.pallas import tpu as pltpu
from jax.experimental.pallas import tpu_sc as plsc

_VB = 4096
_NBUF = 4


def _gather_fn(nc, ns, b_per_w, table_hbm, idx_hbm, out_hbm, idx_v, rows_v, sem):
    wid = lax.axis_index("s") * nc + lax.axis_index("c")
    base = wid * b_per_w
    pltpu.sync_copy(idx_hbm.at[pl.ds(base, b_per_w)], idx_v)
    pltpu.async_copy(table_hbm.at[idx_v], rows_v, sem).wait()
    pltpu.sync_copy(rows_v, out_hbm.at[pl.ds(base, b_per_w)])


def _sc_gather(table, idx_flat):
    info = plsc.get_sparse_core_info()
    nc, ns = info.num_cores, info.num_subcores
    nw = nc * ns
    b_total, d = idx_flat.shape[0], table.shape[1]
    b_per_w = b_total // nw
    mesh = plsc.VectorSubcoreMesh(core_axis_name="c", subcore_axis_name="s")
    kern = functools.partial(
        pl.kernel,
        mesh=mesh,
        out_type=jax.ShapeDtypeStruct((b_total, d), jnp.float32),
        scratch_types=[
            pltpu.VMEM((b_per_w,), jnp.int32),
            pltpu.VMEM((b_per_w, d), jnp.float32),
            pltpu.SemaphoreType.DMA,
        ],
        compiler_params=pltpu.CompilerParams(use_tc_tiling_on_sc=False),
    )(functools.partial(_gather_fn, nc, ns, b_per_w))
    return kern(table, idx_flat)


def _main_body(ng, h_ref, w_ref, b_ref, out_hbm, *scratch):
    accs, sems = scratch[:_NBUF], scratch[_NBUF:]
    g = pl.program_id(0)

    for k in range(_NBUF):
        tile = g * _NBUF + k

        @pl.when(g >= 1)
        def _wait_prev(k=k, tile=tile):
            pltpu.make_async_copy(
                accs[k],
                out_hbm.at[:, pl.ds((tile - _NBUF) * _VB, _VB)],
                sems[k],
            ).wait()

        accs[k][...] = (
            jnp.dot(
                h_ref[...],
                w_ref[:, k * _VB : (k + 1) * _VB],
                preferred_element_type=jnp.float32,
            )
            + b_ref[:, k * _VB : (k + 1) * _VB]
        )
        pltpu.make_async_copy(
            accs[k], out_hbm.at[:, pl.ds(tile * _VB, _VB)], sems[k]
        ).start()

    @pl.when(g == ng - 1)
    def _drain():
        for k in range(_NBUF):
            tile = (ng - 1) * _NBUF + k
            pltpu.make_async_copy(
                accs[k], out_hbm.at[:, pl.ds(tile * _VB, _VB)], sems[k]
            ).wait()


def _tail_body(h_ref, w_ref, b_ref, big_ref, o_ref):
    del big_ref
    o_ref[...] = (
        jnp.dot(h_ref[...], w_ref[...], preferred_element_type=jnp.float32)
        + b_ref[...]
    )


def _tc_project(hidden, proj_w, proj_b2d):
    r, h = hidden.shape
    v = proj_w.shape[1]
    n_tiles = v // _VB
    ng_main = n_tiles // _NBUF
    wb = _NBUF * _VB
    main = pl.pallas_call(
        functools.partial(_main_body, ng_main),
        grid=(ng_main,),
        in_specs=[
            pl.BlockSpec((r, h), lambda g: (0, 0)),
            pl.BlockSpec((h, wb), lambda g: (0, g)),
            pl.BlockSpec((1, wb), lambda g: (0, g)),
        ],
        out_specs=pl.BlockSpec(memory_space=pl.ANY),
        out_shape=jax.ShapeDtypeStruct((r, v), jnp.float32),
        scratch_shapes=(
            [pltpu.VMEM((r, _VB), jnp.float32) for _ in range(_NBUF)]
            + [pltpu.SemaphoreType.DMA for _ in range(_NBUF)]
        ),
    )(hidden, proj_w, proj_b2d)
    return pl.pallas_call(
        _tail_body,
        grid=(1,),
        in_specs=[
            pl.BlockSpec((r, h), lambda g: (0, 0)),
            pl.BlockSpec((h, _VB), lambda g: (0, n_tiles)),
            pl.BlockSpec((1, _VB), lambda g: (0, n_tiles)),
            pl.BlockSpec(memory_space=pl.ANY),
        ],
        out_specs=pl.BlockSpec((r, _VB), lambda g: (0, n_tiles)),
        out_shape=jax.ShapeDtypeStruct((r, v), jnp.float32),
        input_output_aliases={3: 0},
    )(hidden, proj_w, proj_b2d, main)


def kernel(input_ids, emb_table, proj_w, proj_b):
    b, q = input_ids.shape
    v = proj_w.shape[1]
    idx_flat = input_ids.reshape(b * q).astype(jnp.int32)
    hidden = _sc_gather(emb_table, idx_flat)
    logits = _tc_project(hidden, proj_w, proj_b.reshape(1, v))
    return logits.reshape(b, q, v)
